# speedup vs baseline: 1.1955x; 1.1955x over previous
import functools

import jax
import jax.numpy as jnp
from jax.experimental import pallas as pl
from jax.experimental.pallas import tpu as pltpu

N = 10000
E = 320000
D = 128
DEPTH = 4
NP = 10240
W = 144
BE = 1000
BN = 512


def _silu(a):
    return a * jax.nn.sigmoid(a)


def _edge_body(ag_ref, bg_ref, wr_ref, be1_ref, We2_ref, be2_ref,
               Wc1_ref, bc1_ref, wc2_ref, out_ref):
    ag = ag_ref[...]
    bg = bg_ref[...]
    x16 = ag[:, 128:] - bg[:, 128:]
    radial = jnp.sum(x16 * x16, axis=1, keepdims=True)
    inv = 1.0 / (jnp.sqrt(radial) + 1e-30)
    p = ag[:, :128] + bg[:, :128] + radial * wr_ref[...] + be1_ref[...]
    z1 = _silu(p)
    msg_h = _silu(jnp.dot(z1, We2_ref[...],
                          preferred_element_type=jnp.float32) + be2_ref[...])
    t = _silu(jnp.dot(msg_h, Wc1_ref[...],
                      preferred_element_type=jnp.float32) + bc1_ref[...])
    coef = jnp.sum(t * wc2_ref[...], axis=1, keepdims=True)
    lane = jax.lax.broadcasted_iota(jnp.int32, x16.shape, 1)
    msg16 = x16 * (coef * inv) + (lane == 3).astype(jnp.float32)
    out_ref[:, :128] = msg_h
    out_ref[:, 128:] = msg16


def _edge_stage(ag, bg, wr, be1, We2, be2, Wc1, bc1, wc2):
    grid = (E // BE,)
    row = lambda i: (i, 0)
    full = lambda i: (0, 0)
    return pl.pallas_call(
        _edge_body,
        grid=grid,
        in_specs=[
            pl.BlockSpec((BE, W), row),
            pl.BlockSpec((BE, W), row),
            pl.BlockSpec((1, 128), full),
            pl.BlockSpec((1, 128), full),
            pl.BlockSpec((128, 128), full),
            pl.BlockSpec((1, 128), full),
            pl.BlockSpec((128, 128), full),
            pl.BlockSpec((1, 128), full),
            pl.BlockSpec((1, 128), full),
        ],
        out_specs=pl.BlockSpec((BE, W), row),
        out_shape=jax.ShapeDtypeStruct((E, W), jnp.float32),
    )(ag, bg, wr, be1, We2, be2, Wc1, bc1, wc2)


def _xnew16(a_ref, p0_ref, p1_ref):
    s = p0_ref[...] + p1_ref[...]
    s16 = s[:, 128:]
    deg = s16[:, 3:4]
    xn16 = jnp.where(deg > 0, s16 / jnp.maximum(deg, 1.0), 0.0)
    lane = jax.lax.broadcasted_iota(jnp.int32, xn16.shape, 1)
    xn16 = jnp.where(lane < 3, xn16, 0.0)
    return s[:, :128], a_ref[:, 128:] + xn16


def _node_mid_body(h_ref, a_ref, p0_ref, p1_ref, Wn1a_ref, Wn1b_ref, bn1_ref,
                   Wn2_ref, bn2_ref, Wea_ref, Web_ref,
                   hout_ref, aout_ref, bout_ref):
    hn, xnew16 = _xnew16(a_ref, p0_ref, p1_ref)
    h = h_ref[...]
    z = _silu(jnp.dot(h, Wn1a_ref[...], preferred_element_type=jnp.float32)
              + jnp.dot(hn, Wn1b_ref[...], preferred_element_type=jnp.float32)
              + bn1_ref[...])
    hnew = jnp.dot(z, Wn2_ref[...], preferred_element_type=jnp.float32) + bn2_ref[...]
    hout_ref[...] = hnew
    aout_ref[:, :128] = jnp.dot(hnew, Wea_ref[...], preferred_element_type=jnp.float32)
    aout_ref[:, 128:] = xnew16
    bout_ref[:, :128] = jnp.dot(hnew, Web_ref[...], preferred_element_type=jnp.float32)
    bout_ref[:, 128:] = xnew16


def _node_mid(h, a, p0, p1, Wn1a, Wn1b, bn1, Wn2, bn2, Wea, Web):
    grid = (NP // BN,)
    row = lambda i: (i, 0)
    full = lambda i: (0, 0)
    return pl.pallas_call(
        _node_mid_body,
        grid=grid,
        in_specs=[
            pl.BlockSpec((BN, 128), row),
            pl.BlockSpec((BN, W), row),
            pl.BlockSpec((BN, W), row),
            pl.BlockSpec((BN, W), row),
            pl.BlockSpec((128, 128), full),
            pl.BlockSpec((128, 128), full),
            pl.BlockSpec((1, 128), full),
            pl.BlockSpec((128, 128), full),
            pl.BlockSpec((1, 128), full),
            pl.BlockSpec((128, 128), full),
            pl.BlockSpec((128, 128), full),
        ],
        out_specs=[
            pl.BlockSpec((BN, 128), row),
            pl.BlockSpec((BN, W), row),
            pl.BlockSpec((BN, W), row),
        ],
        out_shape=[
            jax.ShapeDtypeStruct((NP, 128), jnp.float32),
            jax.ShapeDtypeStruct((NP, W), jnp.float32),
            jax.ShapeDtypeStruct((NP, W), jnp.float32),
        ],
    )(h, a, p0, p1, Wn1a, Wn1b, bn1, Wn2, bn2, Wea, Web)


def _node_last_body(a_ref, p0_ref, p1_ref, xout_ref):
    _, xnew16 = _xnew16(a_ref, p0_ref, p1_ref)
    xout_ref[...] = xnew16


def _node_last(a, p0, p1):
    grid = (NP // BN,)
    row = lambda i: (i, 0)
    return pl.pallas_call(
        _node_last_body,
        grid=grid,
        in_specs=[
            pl.BlockSpec((BN, W), row),
            pl.BlockSpec((BN, W), row),
            pl.BlockSpec((BN, W), row),
        ],
        out_specs=pl.BlockSpec((BN, 16), row),
        out_shape=jax.ShapeDtypeStruct((NP, 16), jnp.float32),
    )(a, p0, p1)


def _init_body(h_ref, x16_ref, Wea_ref, Web_ref, aout_ref, bout_ref):
    h = h_ref[...]
    x16 = x16_ref[...]
    aout_ref[:, :128] = jnp.dot(h, Wea_ref[...], preferred_element_type=jnp.float32)
    aout_ref[:, 128:] = x16
    bout_ref[:, :128] = jnp.dot(h, Web_ref[...], preferred_element_type=jnp.float32)
    bout_ref[:, 128:] = x16


def _init_tables(h, x16, Wea, Web):
    grid = (NP // BN,)
    row = lambda i: (i, 0)
    full = lambda i: (0, 0)
    return pl.pallas_call(
        _init_body,
        grid=grid,
        in_specs=[
            pl.BlockSpec((BN, 128), row),
            pl.BlockSpec((BN, 16), row),
            pl.BlockSpec((128, 128), full),
            pl.BlockSpec((128, 128), full),
        ],
        out_specs=[
            pl.BlockSpec((BN, W), row),
            pl.BlockSpec((BN, W), row),
        ],
        out_shape=[
            jax.ShapeDtypeStruct((NP, W), jnp.float32),
            jax.ShapeDtypeStruct((NP, W), jnp.float32),
        ],
    )(h, x16, Wea, Web)


def _gather_stage(a, b, src, dst):
    return jnp.take(a, src, axis=0), jnp.take(b, dst, axis=0)


def _scatter_stage(msg, dst):
    p0 = jax.ops.segment_sum(msg, dst, num_segments=NP)
    p1 = jnp.zeros_like(p0)
    return p0, p1


def kernel(h, x, edge_index, We1, be1, We2, be2, Wc1, bc1, Wc2, Wn1, bn1, Wn2, bn2):
    src = edge_index[0]
    dst = edge_index[1]
    hp = jnp.pad(h, ((0, NP - N), (0, 0)))
    x16 = jnp.pad(x, ((0, NP - N), (0, 13)))

    a, b = _init_tables(hp, x16, We1[0][:D], We1[0][D:2 * D])
    for l in range(DEPTH):
        ag, bg = _gather_stage(a, b, src, dst)
        msg = _edge_stage(ag, bg, We1[l][2 * D][None, :], be1[l][None, :],
                          We2[l], be2[l][None, :], Wc1[l], bc1[l][None, :],
                          Wc2[l][:, 0][None, :])
        p0, p1 = _scatter_stage(msg, dst)
        if l < DEPTH - 1:
            hp, a, b = _node_mid(hp, a, p0, p1,
                                 Wn1[l][:D], Wn1[l][D:], bn1[l][None, :],
                                 Wn2[l], bn2[l][None, :],
                                 We1[l + 1][:D], We1[l + 1][D:2 * D])
        else:
            xfin = _node_last(a, p0, p1)
    return xfin[:N, :3]


# speedup vs baseline: 1.9757x; 1.6526x over previous
import jax
import jax.numpy as jnp
from jax.experimental import pallas as pl
from jax.experimental.pallas import tpu as pltpu
from jax.experimental.pallas import tpu_sc as plsc

N = 10000
E = 320000
D = 128
DEPTH = 4
NP = 10240
EP = 327680
TW = 256
BE = 1024
BN = 1024

NSC = 2
NTEC = 16
NWORK = NSC * NTEC
EW = EP // NWORK
GCH = 128
GNCH = EW // GCH
EWS = EP // NTEC
SNCH = EWS // GCH
RPT = NP // NTEC

_sc_mesh = plsc.VectorSubcoreMesh(
    core_axis_name="c", subcore_axis_name="s", num_cores=NSC, num_subcores=NTEC)


def _silu(a):
    return a * jax.nn.sigmoid(a)


def _gather_body(a_hbm, b_hbm, src_hbm, dst_hbm, ag_out, bg_out,
                 srcv, dstv, bufa, bufb, sem):
    c = jax.lax.axis_index("c")
    s = jax.lax.axis_index("s")
    w = c * NTEC + s
    pltpu.sync_copy(src_hbm.at[w], srcv)
    pltpu.sync_copy(dst_hbm.at[w], dstv)

    def step(j, carry):
        base = pl.multiple_of(w * EW + j * GCH, 8)
        cpa = pltpu.async_copy(a_hbm.at[srcv.at[j]], bufa, sem)
        cpb = pltpu.async_copy(b_hbm.at[dstv.at[j]], bufb, sem)
        cpa.wait()
        cpb.wait()
        pltpu.sync_copy(bufa, ag_out.at[pl.ds(base, GCH)])
        pltpu.sync_copy(bufb, bg_out.at[pl.ds(base, GCH)])
        return carry

    jax.lax.fori_loop(0, GNCH, step, 0)


_gather_call = pl.kernel(
    _gather_body,
    out_type=[jax.ShapeDtypeStruct((EP, TW), jnp.float32),
              jax.ShapeDtypeStruct((EP, TW), jnp.float32)],
    mesh=_sc_mesh,
    scratch_types=[
        pltpu.VMEM((GNCH, GCH), jnp.int32),
        pltpu.VMEM((GNCH, GCH), jnp.int32),
        pltpu.VMEM((GCH, TW), jnp.float32),
        pltpu.VMEM((GCH, TW), jnp.float32),
        pltpu.SemaphoreType.DMA,
    ],
)


def _scatter_body(msg_hbm, msgx_hbm, dst_hbm, zeros_hbm, hn_out, xn_out,
                  dstv, mbuf, acc, sem):
    c = jax.lax.axis_index("c")
    s = jax.lax.axis_index("s")
    pltpu.sync_copy(dst_hbm.at[s], dstv)
    r0 = s * RPT
    pltpu.sync_copy(zeros_hbm.at[pl.ds(r0, RPT)], acc.at[pl.ds(r0, RPT)])
    plsc.subcore_barrier()

    def step(j, carry):
        base = pl.multiple_of(s * EWS + j * GCH, 8)
        idxd = dstv.at[j]

        @pl.when(c == 0)
        def _():
            pltpu.async_copy(msg_hbm.at[pl.ds(base, GCH)], mbuf, sem).wait()
            pltpu.sync_copy(mbuf, acc.at[idxd], add=True)

        @pl.when(c == 1)
        def _():
            pltpu.async_copy(msgx_hbm.at[pl.ds(base, GCH)], mbuf, sem).wait()
            pltpu.sync_copy(mbuf, acc.at[idxd], add=True)

        return carry

    jax.lax.fori_loop(0, SNCH, step, 0)
    plsc.subcore_barrier()

    @pl.when(c == 0)
    def _():
        pltpu.sync_copy(acc.at[pl.ds(r0, RPT)], hn_out.at[pl.ds(r0, RPT)])

    @pl.when(c == 1)
    def _():
        pltpu.sync_copy(acc.at[pl.ds(r0, RPT)], xn_out.at[pl.ds(r0, RPT)])


_scatter_call = pl.kernel(
    _scatter_body,
    out_type=[jax.ShapeDtypeStruct((NP, 128), jnp.float32),
              jax.ShapeDtypeStruct((NP, 128), jnp.float32)],
    mesh=_sc_mesh,
    scratch_types=[
        pltpu.VMEM((SNCH, GCH), jnp.int32),
        pltpu.VMEM((GCH, 128), jnp.float32),
        pltpu.VMEM_SHARED((NP, 128), jnp.float32),
        pltpu.SemaphoreType.DMA,
    ],
)


def _edge_body(ag_ref, bg_ref, wr_ref, We2_ref, be2_ref,
               Wc1_ref, bc1_ref, wc2_ref, msg_ref, msgx_ref):
    ag = ag_ref[...]
    bg = bg_ref[...]
    xdiff = ag[:, 128:] - bg[:, 128:]
    radial = jnp.sum(xdiff * xdiff, axis=1, keepdims=True)
    inv = 1.0 / (jnp.sqrt(radial) + 1e-30)
    p = ag[:, :128] + bg[:, :128] + radial * wr_ref[...]
    z1 = _silu(p)
    msg_h = _silu(jnp.dot(z1, We2_ref[...],
                          preferred_element_type=jnp.float32) + be2_ref[...])
    t = _silu(jnp.dot(msg_h, Wc1_ref[...],
                      preferred_element_type=jnp.float32) + bc1_ref[...])
    coef = jnp.sum(t * wc2_ref[...], axis=1, keepdims=True)
    lane = jax.lax.broadcasted_iota(jnp.int32, xdiff.shape, 1)
    msg_ref[...] = msg_h
    msgx_ref[...] = xdiff * (coef * inv) + (lane == 3).astype(jnp.float32)


def _edge_stage(ag, bg, wr, We2, be2, Wc1, bc1, wc2):
    grid = (EP // BE,)
    row = lambda i: (i, 0)
    full = lambda i: (0, 0)
    return pl.pallas_call(
        _edge_body,
        grid=grid,
        in_specs=[
            pl.BlockSpec((BE, TW), row),
            pl.BlockSpec((BE, TW), row),
            pl.BlockSpec((1, 128), full),
            pl.BlockSpec((128, 128), full),
            pl.BlockSpec((1, 128), full),
            pl.BlockSpec((128, 128), full),
            pl.BlockSpec((1, 128), full),
            pl.BlockSpec((1, 128), full),
        ],
        out_specs=[
            pl.BlockSpec((BE, 128), row),
            pl.BlockSpec((BE, 128), row),
        ],
        out_shape=[
            jax.ShapeDtypeStruct((EP, 128), jnp.float32),
            jax.ShapeDtypeStruct((EP, 128), jnp.float32),
        ],
    )(ag, bg, wr, We2, be2, Wc1, bc1, wc2)


def _xnew(a_ref, xn_ref):
    xn = xn_ref[...]
    deg = xn[:, 3:4]
    upd = jnp.where(deg > 0, xn / jnp.maximum(deg, 1.0), 0.0)
    lane = jax.lax.broadcasted_iota(jnp.int32, xn.shape, 1)
    upd = jnp.where(lane < 3, upd, 0.0)
    return a_ref[:, 128:] + upd


def _node_mid_body(h_ref, hn_ref, xn_ref, a_ref,
                   Wn1a_ref, Wn1b_ref, bn1_ref, Wn2_ref, bn2_ref,
                   Wea_ref, Web_ref, be1n_ref,
                   hout_ref, aout_ref, bout_ref):
    x128 = _xnew(a_ref, xn_ref)
    h = h_ref[...]
    z = _silu(jnp.dot(h, Wn1a_ref[...], preferred_element_type=jnp.float32)
              + jnp.dot(hn_ref[...], Wn1b_ref[...],
                        preferred_element_type=jnp.float32)
              + bn1_ref[...])
    hnew = jnp.dot(z, Wn2_ref[...], preferred_element_type=jnp.float32) + bn2_ref[...]
    hout_ref[...] = hnew
    aout_ref[:, :128] = jnp.dot(hnew, Wea_ref[...],
                                preferred_element_type=jnp.float32)
    aout_ref[:, 128:] = x128
    bout_ref[:, :128] = jnp.dot(hnew, Web_ref[...],
                                preferred_element_type=jnp.float32) + be1n_ref[...]
    bout_ref[:, 128:] = x128


def _node_mid(h, hn, xn, a, Wn1a, Wn1b, bn1, Wn2, bn2, Wea, Web, be1n):
    grid = (NP // BN,)
    row = lambda i: (i, 0)
    full = lambda i: (0, 0)
    return pl.pallas_call(
        _node_mid_body,
        grid=grid,
        in_specs=[
            pl.BlockSpec((BN, 128), row),
            pl.BlockSpec((BN, 128), row),
            pl.BlockSpec((BN, 128), row),
            pl.BlockSpec((BN, TW), row),
            pl.BlockSpec((128, 128), full),
            pl.BlockSpec((128, 128), full),
            pl.BlockSpec((1, 128), full),
            pl.BlockSpec((128, 128), full),
            pl.BlockSpec((1, 128), full),
            pl.BlockSpec((128, 128), full),
            pl.BlockSpec((128, 128), full),
            pl.BlockSpec((1, 128), full),
        ],
        out_specs=[
            pl.BlockSpec((BN, 128), row),
            pl.BlockSpec((BN, TW), row),
            pl.BlockSpec((BN, TW), row),
        ],
        out_shape=[
            jax.ShapeDtypeStruct((NP, 128), jnp.float32),
            jax.ShapeDtypeStruct((NP, TW), jnp.float32),
            jax.ShapeDtypeStruct((NP, TW), jnp.float32),
        ],
    )(h, hn, xn, a, Wn1a, Wn1b, bn1, Wn2, bn2, Wea, Web, be1n)


def _node_last_body(xn_ref, a_ref, xout_ref):
    xout_ref[...] = _xnew(a_ref, xn_ref)


def _node_last(xn, a):
    grid = (NP // BN,)
    row = lambda i: (i, 0)
    return pl.pallas_call(
        _node_last_body,
        grid=grid,
        in_specs=[
            pl.BlockSpec((BN, 128), row),
            pl.BlockSpec((BN, TW), row),
        ],
        out_specs=pl.BlockSpec((BN, 128), row),
        out_shape=jax.ShapeDtypeStruct((NP, 128), jnp.float32),
    )(xn, a)


def _init_body(h_ref, x128_ref, Wea_ref, Web_ref, be1n_ref,
               aout_ref, bout_ref):
    h = h_ref[...]
    x128 = x128_ref[...]
    aout_ref[:, :128] = jnp.dot(h, Wea_ref[...],
                                preferred_element_type=jnp.float32)
    aout_ref[:, 128:] = x128
    bout_ref[:, :128] = jnp.dot(h, Web_ref[...],
                                preferred_element_type=jnp.float32) + be1n_ref[...]
    bout_ref[:, 128:] = x128


def _init_tables(h, x128, Wea, Web, be1n):
    grid = (NP // BN,)
    row = lambda i: (i, 0)
    full = lambda i: (0, 0)
    return pl.pallas_call(
        _init_body,
        grid=grid,
        in_specs=[
            pl.BlockSpec((BN, 128), row),
            pl.BlockSpec((BN, 128), row),
            pl.BlockSpec((128, 128), full),
            pl.BlockSpec((128, 128), full),
            pl.BlockSpec((1, 128), full),
        ],
        out_specs=[
            pl.BlockSpec((BN, TW), row),
            pl.BlockSpec((BN, TW), row),
        ],
        out_shape=[
            jax.ShapeDtypeStruct((NP, TW), jnp.float32),
            jax.ShapeDtypeStruct((NP, TW), jnp.float32),
        ],
    )(h, x128, Wea, Web, be1n)


def kernel(h, x, edge_index, We1, be1, We2, be2, Wc1, bc1, Wc2, Wn1, bn1, Wn2, bn2):
    src = edge_index[0]
    dst = edge_index[1]
    srcp = jnp.concatenate([src, jnp.zeros((EP - E,), src.dtype)])
    dstp = jnp.concatenate([dst, jnp.full((EP - E,), NP - 1, dst.dtype)])
    src3 = srcp.reshape(NWORK, GNCH, GCH)
    dst3 = dstp.reshape(NWORK, GNCH, GCH)
    dst2 = dstp.reshape(NTEC, SNCH, GCH)
    zerorow = jnp.zeros((NP, 128), jnp.float32)
    hp = jnp.pad(h, ((0, NP - N), (0, 0)))
    x128 = jnp.pad(x, ((0, NP - N), (0, 125)))

    a, b = _init_tables(hp, x128, We1[0][:D], We1[0][D:2 * D],
                        be1[0][None, :])
    for l in range(DEPTH):
        ag, bg = _gather_call(a, b, src3, dst3)
        msg, msgx = _edge_stage(ag, bg, We1[l][2 * D][None, :],
                                We2[l], be2[l][None, :],
                                Wc1[l], bc1[l][None, :], Wc2[l][:, 0][None, :])
        hn, xn = _scatter_call(msg, msgx, dst2, zerorow)
        if l < DEPTH - 1:
            hp, a, b = _node_mid(hp, hn, xn, a,
                                 Wn1[l][:D], Wn1[l][D:], bn1[l][None, :],
                                 Wn2[l], bn2[l][None, :],
                                 We1[l + 1][:D], We1[l + 1][D:2 * D],
                                 be1[l + 1][None, :])
        else:
            xfin = _node_last(xn, a)
    return xfin[:N, :3]


# speedup vs baseline: 2.1063x; 1.0661x over previous
import jax
import jax.numpy as jnp
from jax.experimental import pallas as pl
from jax.experimental.pallas import tpu as pltpu
from jax.experimental.pallas import tpu_sc as plsc

N = 10000
E = 320000
D = 128
DEPTH = 4
NP = 10240
EP = 327680
TW = 256
BE = 1024
BN = 1024

NSC = 2
NTEC = 16
NWORK = NSC * NTEC
EW = EP // NWORK
GCH = 80
GNCH = EW // GCH
EWS = EP // NTEC
SCH = 128
SNCH = EWS // SCH
SHALF = SNCH // 2
RPT = NP // NTEC

_sc_mesh = plsc.VectorSubcoreMesh(
    core_axis_name="c", subcore_axis_name="s", num_cores=NSC, num_subcores=NTEC)


def _silu(a):
    return a * jax.nn.sigmoid(a)


def _gather_body(a_hbm, b_hbm, src_hbm, dst_hbm, ag_out, bg_out,
                 srcv, dstv, bufa0, bufb0, bufa1, bufb1, sem0, sem1):
    c = jax.lax.axis_index("c")
    s = jax.lax.axis_index("s")
    w = c * NTEC + s
    pltpu.sync_copy(src_hbm.at[w], srcv)
    pltpu.sync_copy(dst_hbm.at[w], dstv)
    bufs = ((bufa0, bufb0, sem0), (bufa1, bufb1, sem1))

    def pair(t, carry):
        cps = []
        for p in range(2):
            j = t * 2 + p
            ba, bb, sm = bufs[p]
            cps.append((pltpu.async_copy(a_hbm.at[srcv.at[j]], ba, sm),
                        pltpu.async_copy(b_hbm.at[dstv.at[j]], bb, sm)))
        for p in range(2):
            j = t * 2 + p
            base = pl.multiple_of(w * EW + j * GCH, 8)
            ba, bb, _ = bufs[p]
            cpa, cpb = cps[p]
            cpa.wait()
            cpb.wait()
            pltpu.sync_copy(ba, ag_out.at[pl.ds(base, GCH)])
            pltpu.sync_copy(bb, bg_out.at[pl.ds(base, GCH)])
        return carry

    jax.lax.fori_loop(0, GNCH // 2, pair, 0)


_gather_call = pl.kernel(
    _gather_body,
    out_type=[jax.ShapeDtypeStruct((EP, TW), jnp.float32),
              jax.ShapeDtypeStruct((EP, TW), jnp.float32)],
    mesh=_sc_mesh,
    scratch_types=[
        pltpu.VMEM((GNCH, GCH), jnp.int32),
        pltpu.VMEM((GNCH, GCH), jnp.int32),
        pltpu.VMEM((GCH, TW), jnp.float32),
        pltpu.VMEM((GCH, TW), jnp.float32),
        pltpu.VMEM((GCH, TW), jnp.float32),
        pltpu.VMEM((GCH, TW), jnp.float32),
        pltpu.SemaphoreType.DMA,
        pltpu.SemaphoreType.DMA,
    ],
)


def _scatter_body(msgall_hbm, dst_hbm, zeros_hbm, hn_out, xn_out,
                  dstv, mbuf0, mbuf1, acc, sem0, sem1):
    c = jax.lax.axis_index("c")
    s = jax.lax.axis_index("s")
    r0 = s * RPT
    pltpu.sync_copy(zeros_hbm.at[pl.ds(r0, RPT)], acc.at[pl.ds(r0, RPT)])
    plsc.subcore_barrier()
    bufs = ((mbuf0, sem0), (mbuf1, sem1))

    for half in range(2):
        pltpu.sync_copy(dst_hbm.at[s].at[pl.ds(half * SHALF, SHALF)], dstv)

        def pair(t, carry):
            cps = []
            for p in range(2):
                j = t * 2 + p
                base = pl.multiple_of(
                    c * EP + s * EWS + (half * SHALF + j) * SCH, 8)
                mb, sm = bufs[p]
                cps.append(pltpu.async_copy(msgall_hbm.at[pl.ds(base, SCH)],
                                            mb, sm))
            for p in range(2):
                j = t * 2 + p
                mb, _ = bufs[p]
                cps[p].wait()
                pltpu.sync_copy(mb, acc.at[dstv.at[j]], add=True)
            return carry

        jax.lax.fori_loop(0, SHALF // 2, pair, 0)

    plsc.subcore_barrier()

    @pl.when(c == 0)
    def _():
        pltpu.sync_copy(acc.at[pl.ds(r0, RPT)], hn_out.at[pl.ds(r0, RPT)])

    @pl.when(c == 1)
    def _():
        pltpu.sync_copy(acc.at[pl.ds(r0, RPT)], xn_out.at[pl.ds(r0, RPT)])


_scatter_call = pl.kernel(
    _scatter_body,
    out_type=[jax.ShapeDtypeStruct((NP, 128), jnp.float32),
              jax.ShapeDtypeStruct((NP, 128), jnp.float32)],
    mesh=_sc_mesh,
    scratch_types=[
        pltpu.VMEM((SHALF, SCH), jnp.int32),
        pltpu.VMEM((SCH, 128), jnp.float32),
        pltpu.VMEM((SCH, 128), jnp.float32),
        pltpu.VMEM_SHARED((NP, 128), jnp.float32),
        pltpu.SemaphoreType.DMA,
        pltpu.SemaphoreType.DMA,
    ],
)


def _edge_body(ag_ref, bg_ref, wr_ref, We2_ref, be2_ref,
               Wc1_ref, bc1_ref, wc2_ref, msgall_ref):
    ag = ag_ref[...]
    bg = bg_ref[...]
    xdiff = ag[:, 128:] - bg[:, 128:]
    radial = jnp.sum(xdiff * xdiff, axis=1, keepdims=True)
    inv = 1.0 / (jnp.sqrt(radial) + 1e-30)
    p = ag[:, :128] + bg[:, :128] + radial * wr_ref[...]
    z1 = _silu(p)
    msg_h = _silu(jnp.dot(z1, We2_ref[...],
                          preferred_element_type=jnp.float32) + be2_ref[...])
    t = _silu(jnp.dot(msg_h, Wc1_ref[...],
                      preferred_element_type=jnp.float32) + bc1_ref[...])
    coef = jnp.sum(t * wc2_ref[...], axis=1, keepdims=True)
    lane = jax.lax.broadcasted_iota(jnp.int32, xdiff.shape, 1)
    msgall_ref[0] = msg_h
    msgall_ref[1] = xdiff * (coef * inv) + (lane == 3).astype(jnp.float32)


def _edge_stage(ag, bg, wr, We2, be2, Wc1, bc1, wc2):
    grid = (EP // BE,)
    row = lambda i: (i, 0)
    full = lambda i: (0, 0)
    return pl.pallas_call(
        _edge_body,
        grid=grid,
        in_specs=[
            pl.BlockSpec((BE, TW), row),
            pl.BlockSpec((BE, TW), row),
            pl.BlockSpec((1, 128), full),
            pl.BlockSpec((128, 128), full),
            pl.BlockSpec((1, 128), full),
            pl.BlockSpec((128, 128), full),
            pl.BlockSpec((1, 128), full),
            pl.BlockSpec((1, 128), full),
        ],
        out_specs=pl.BlockSpec((2, BE, 128), lambda i: (0, i, 0)),
        out_shape=jax.ShapeDtypeStruct((2, EP, 128), jnp.float32),
    )(ag, bg, wr, We2, be2, Wc1, bc1, wc2)


def _xnew(a_ref, xn_ref):
    xn = xn_ref[...]
    deg = xn[:, 3:4]
    upd = jnp.where(deg > 0, xn / jnp.maximum(deg, 1.0), 0.0)
    lane = jax.lax.broadcasted_iota(jnp.int32, xn.shape, 1)
    upd = jnp.where(lane < 3, upd, 0.0)
    return a_ref[:, 128:] + upd


def _node_mid_body(h_ref, hn_ref, xn_ref, a_ref,
                   Wn1a_ref, Wn1b_ref, bn1_ref, Wn2_ref, bn2_ref,
                   Wea_ref, Web_ref, be1n_ref,
                   hout_ref, aout_ref, bout_ref):
    x128 = _xnew(a_ref, xn_ref)
    h = h_ref[...]
    z = _silu(jnp.dot(h, Wn1a_ref[...], preferred_element_type=jnp.float32)
              + jnp.dot(hn_ref[...], Wn1b_ref[...],
                        preferred_element_type=jnp.float32)
              + bn1_ref[...])
    hnew = jnp.dot(z, Wn2_ref[...], preferred_element_type=jnp.float32) + bn2_ref[...]
    hout_ref[...] = hnew
    aout_ref[:, :128] = jnp.dot(hnew, Wea_ref[...],
                                preferred_element_type=jnp.float32)
    aout_ref[:, 128:] = x128
    bout_ref[:, :128] = jnp.dot(hnew, Web_ref[...],
                                preferred_element_type=jnp.float32) + be1n_ref[...]
    bout_ref[:, 128:] = x128


def _node_mid(h, hn, xn, a, Wn1a, Wn1b, bn1, Wn2, bn2, Wea, Web, be1n):
    grid = (NP // BN,)
    row = lambda i: (i, 0)
    full = lambda i: (0, 0)
    return pl.pallas_call(
        _node_mid_body,
        grid=grid,
        in_specs=[
            pl.BlockSpec((BN, 128), row),
            pl.BlockSpec((BN, 128), row),
            pl.BlockSpec((BN, 128), row),
            pl.BlockSpec((BN, TW), row),
            pl.BlockSpec((128, 128), full),
            pl.BlockSpec((128, 128), full),
            pl.BlockSpec((1, 128), full),
            pl.BlockSpec((128, 128), full),
            pl.BlockSpec((1, 128), full),
            pl.BlockSpec((128, 128), full),
            pl.BlockSpec((128, 128), full),
            pl.BlockSpec((1, 128), full),
        ],
        out_specs=[
            pl.BlockSpec((BN, 128), row),
            pl.BlockSpec((BN, TW), row),
            pl.BlockSpec((BN, TW), row),
        ],
        out_shape=[
            jax.ShapeDtypeStruct((NP, 128), jnp.float32),
            jax.ShapeDtypeStruct((NP, TW), jnp.float32),
            jax.ShapeDtypeStruct((NP, TW), jnp.float32),
        ],
    )(h, hn, xn, a, Wn1a, Wn1b, bn1, Wn2, bn2, Wea, Web, be1n)


def _node_last_body(xn_ref, a_ref, xout_ref):
    xout_ref[...] = _xnew(a_ref, xn_ref)


def _node_last(xn, a):
    grid = (NP // BN,)
    row = lambda i: (i, 0)
    return pl.pallas_call(
        _node_last_body,
        grid=grid,
        in_specs=[
            pl.BlockSpec((BN, 128), row),
            pl.BlockSpec((BN, TW), row),
        ],
        out_specs=pl.BlockSpec((BN, 128), row),
        out_shape=jax.ShapeDtypeStruct((NP, 128), jnp.float32),
    )(xn, a)


def _init_body(h_ref, x128_ref, Wea_ref, Web_ref, be1n_ref,
               aout_ref, bout_ref):
    h = h_ref[...]
    x128 = x128_ref[...]
    aout_ref[:, :128] = jnp.dot(h, Wea_ref[...],
                                preferred_element_type=jnp.float32)
    aout_ref[:, 128:] = x128
    bout_ref[:, :128] = jnp.dot(h, Web_ref[...],
                                preferred_element_type=jnp.float32) + be1n_ref[...]
    bout_ref[:, 128:] = x128


def _init_tables(h, x128, Wea, Web, be1n):
    grid = (NP // BN,)
    row = lambda i: (i, 0)
    full = lambda i: (0, 0)
    return pl.pallas_call(
        _init_body,
        grid=grid,
        in_specs=[
            pl.BlockSpec((BN, 128), row),
            pl.BlockSpec((BN, 128), row),
            pl.BlockSpec((128, 128), full),
            pl.BlockSpec((128, 128), full),
            pl.BlockSpec((1, 128), full),
        ],
        out_specs=[
            pl.BlockSpec((BN, TW), row),
            pl.BlockSpec((BN, TW), row),
        ],
        out_shape=[
            jax.ShapeDtypeStruct((NP, TW), jnp.float32),
            jax.ShapeDtypeStruct((NP, TW), jnp.float32),
        ],
    )(h, x128, Wea, Web, be1n)


def kernel(h, x, edge_index, We1, be1, We2, be2, Wc1, bc1, Wc2, Wn1, bn1, Wn2, bn2):
    src = edge_index[0]
    dst = edge_index[1]
    srcp = jnp.concatenate([src, jnp.zeros((EP - E,), src.dtype)])
    dstp = jnp.concatenate([dst, jnp.full((EP - E,), NP - 1, dst.dtype)])
    src3 = srcp.reshape(NWORK, GNCH, GCH)
    dst3 = dstp.reshape(NWORK, GNCH, GCH)
    dst2 = dstp.reshape(NTEC, SNCH, SCH)
    zerorow = jnp.zeros((NP, 128), jnp.float32)
    hp = jnp.pad(h, ((0, NP - N), (0, 0)))
    x128 = jnp.pad(x, ((0, NP - N), (0, 125)))

    a, b = _init_tables(hp, x128, We1[0][:D], We1[0][D:2 * D],
                        be1[0][None, :])
    for l in range(DEPTH):
        ag, bg = _gather_call(a, b, src3, dst3)
        msgall = _edge_stage(ag, bg, We1[l][2 * D][None, :],
                             We2[l], be2[l][None, :],
                             Wc1[l], bc1[l][None, :], Wc2[l][:, 0][None, :])
        hn, xn = _scatter_call(msgall.reshape(2 * EP, 128), dst2, zerorow)
        if l < DEPTH - 1:
            hp, a, b = _node_mid(hp, hn, xn, a,
                                 Wn1[l][:D], Wn1[l][D:], bn1[l][None, :],
                                 Wn2[l], bn2[l][None, :],
                                 We1[l + 1][:D], We1[l + 1][D:2 * D],
                                 be1[l + 1][None, :])
        else:
            xfin = _node_last(xn, a)
    return xfin[:N, :3]


# speedup vs baseline: 2.4155x; 1.1468x over previous
import jax
import jax.numpy as jnp
from jax.experimental import pallas as pl
from jax.experimental.pallas import tpu as pltpu
from jax.experimental.pallas import tpu_sc as plsc

N = 10000
E = 320000
D = 128
DEPTH = 4
NP = 10240
EP = 327680
TW = 256
BE = 1024
BN = 1024

NSC = 2
NTEC = 16
NWORK = NSC * NTEC
EW = EP // NWORK
GCH = 128
GNCH = EW // GCH
EWS = EP // NTEC
SCH = 128
SNCH = EWS // SCH
SHALF = SNCH // 2
RPT = NP // NTEC

_sc_mesh = plsc.VectorSubcoreMesh(
    core_axis_name="c", subcore_axis_name="s", num_cores=NSC, num_subcores=NTEC)


def _silu(a):
    return a * jax.nn.sigmoid(a)


def _pack2(feat, ext):
    fb = jax.lax.shift_right_logical(
        jax.lax.bitcast_convert_type(feat, jnp.uint32) + jnp.uint32(0x8000),
        jnp.uint32(16))
    eb = jax.lax.bitwise_and(
        jax.lax.bitcast_convert_type(ext, jnp.uint32) + jnp.uint32(0x8000),
        jnp.uint32(0xFFFF0000))
    return jax.lax.bitcast_convert_type(jax.lax.bitwise_or(fb, eb),
                                        jnp.float32)


def _unpack_feat(w):
    b = jax.lax.shift_left(jax.lax.bitcast_convert_type(w, jnp.uint32),
                           jnp.uint32(16))
    return jax.lax.bitcast_convert_type(b, jnp.float32)


def _unpack_ext(w):
    b = jax.lax.bitwise_and(jax.lax.bitcast_convert_type(w, jnp.uint32),
                            jnp.uint32(0xFFFF0000))
    return jax.lax.bitcast_convert_type(b, jnp.float32)


def _gather_body(a_hbm, b_hbm, src_hbm, dst_hbm, ag_out, bg_out,
                 srcv, dstv, bufa0, bufb0, bufa1, bufb1, sem0, sem1):
    c = jax.lax.axis_index("c")
    s = jax.lax.axis_index("s")
    w = c * NTEC + s
    pltpu.sync_copy(src_hbm.at[w], srcv)
    pltpu.sync_copy(dst_hbm.at[w], dstv)
    bufs = ((bufa0, bufb0, sem0), (bufa1, bufb1, sem1))

    def pair(t, carry):
        cps = []
        for p in range(2):
            j = t * 2 + p
            ba, bb, sm = bufs[p]
            cps.append((pltpu.async_copy(a_hbm.at[srcv.at[j]], ba, sm),
                        pltpu.async_copy(b_hbm.at[dstv.at[j]], bb, sm)))
        for p in range(2):
            j = t * 2 + p
            base = pl.multiple_of(w * EW + j * GCH, 8)
            ba, bb, _ = bufs[p]
            cpa, cpb = cps[p]
            cpa.wait()
            cpb.wait()
            pltpu.sync_copy(ba, ag_out.at[pl.ds(base, GCH)])
            pltpu.sync_copy(bb, bg_out.at[pl.ds(base, GCH)])
        return carry

    jax.lax.fori_loop(0, GNCH // 2, pair, 0)


_gather_call = pl.kernel(
    _gather_body,
    out_type=[jax.ShapeDtypeStruct((EP, 128), jnp.float32),
              jax.ShapeDtypeStruct((EP, 128), jnp.float32)],
    mesh=_sc_mesh,
    scratch_types=[
        pltpu.VMEM((GNCH, GCH), jnp.int32),
        pltpu.VMEM((GNCH, GCH), jnp.int32),
        pltpu.VMEM((GCH, 128), jnp.float32),
        pltpu.VMEM((GCH, 128), jnp.float32),
        pltpu.VMEM((GCH, 128), jnp.float32),
        pltpu.VMEM((GCH, 128), jnp.float32),
        pltpu.SemaphoreType.DMA,
        pltpu.SemaphoreType.DMA,
    ],
)


def _scatter_body(msgall_hbm, dst_hbm, zeros_hbm, hn_out, xn_out,
                  dstv, mbuf0, mbuf1, acc, sem0, sem1):
    c = jax.lax.axis_index("c")
    s = jax.lax.axis_index("s")
    r0 = s * RPT
    pltpu.sync_copy(zeros_hbm.at[pl.ds(r0, RPT)], acc.at[pl.ds(r0, RPT)])
    plsc.subcore_barrier()
    bufs = ((mbuf0, sem0), (mbuf1, sem1))

    for half in range(2):
        pltpu.sync_copy(dst_hbm.at[s].at[pl.ds(half * SHALF, SHALF)], dstv)

        def pair(t, carry):
            cps = []
            for p in range(2):
                j = t * 2 + p
                base = pl.multiple_of(
                    c * EP + s * EWS + (half * SHALF + j) * SCH, 8)
                mb, sm = bufs[p]
                cps.append(pltpu.async_copy(msgall_hbm.at[pl.ds(base, SCH)],
                                            mb, sm))
            for p in range(2):
                j = t * 2 + p
                mb, _ = bufs[p]
                cps[p].wait()
                pltpu.sync_copy(mb, acc.at[dstv.at[j]], add=True)
            return carry

        jax.lax.fori_loop(0, SHALF // 2, pair, 0)

    plsc.subcore_barrier()

    @pl.when(c == 0)
    def _():
        pltpu.sync_copy(acc.at[pl.ds(r0, RPT)], hn_out.at[pl.ds(r0, RPT)])

    @pl.when(c == 1)
    def _():
        pltpu.sync_copy(acc.at[pl.ds(r0, RPT)], xn_out.at[pl.ds(r0, RPT)])


_scatter_call = pl.kernel(
    _scatter_body,
    out_type=[jax.ShapeDtypeStruct((NP, 128), jnp.float32),
              jax.ShapeDtypeStruct((NP, 128), jnp.float32)],
    mesh=_sc_mesh,
    scratch_types=[
        pltpu.VMEM((SHALF, SCH), jnp.int32),
        pltpu.VMEM((SCH, 128), jnp.float32),
        pltpu.VMEM((SCH, 128), jnp.float32),
        pltpu.VMEM_SHARED((NP, 128), jnp.float32),
        pltpu.SemaphoreType.DMA,
        pltpu.SemaphoreType.DMA,
    ],
)


def _edge_body(ag_ref, bg_ref, wr_ref, We2_ref, be2_ref,
               Wc1_ref, bc1_ref, wc2_ref, msgall_ref):
    ag = ag_ref[...]
    bg = bg_ref[...]
    xdiff = _unpack_ext(ag) - _unpack_ext(bg)
    radial = jnp.sum(xdiff * xdiff, axis=1, keepdims=True)
    inv = 1.0 / (jnp.sqrt(radial) + 1e-30)
    p = _unpack_feat(ag) + _unpack_feat(bg) + radial * wr_ref[...]
    z1 = _silu(p)
    msg_h = _silu(jnp.dot(z1, We2_ref[...],
                          preferred_element_type=jnp.float32) + be2_ref[...])
    t = _silu(jnp.dot(msg_h, Wc1_ref[...],
                      preferred_element_type=jnp.float32) + bc1_ref[...])
    coef = jnp.sum(t * wc2_ref[...], axis=1, keepdims=True)
    lane = jax.lax.broadcasted_iota(jnp.int32, xdiff.shape, 1)
    msgall_ref[0] = msg_h
    msgall_ref[1] = xdiff * (coef * inv) + (lane == 3).astype(jnp.float32)


def _edge_stage(ag, bg, wr, We2, be2, Wc1, bc1, wc2):
    grid = (EP // BE,)
    row = lambda i: (i, 0)
    full = lambda i: (0, 0)
    return pl.pallas_call(
        _edge_body,
        grid=grid,
        in_specs=[
            pl.BlockSpec((BE, 128), row),
            pl.BlockSpec((BE, 128), row),
            pl.BlockSpec((1, 128), full),
            pl.BlockSpec((128, 128), full),
            pl.BlockSpec((1, 128), full),
            pl.BlockSpec((128, 128), full),
            pl.BlockSpec((1, 128), full),
            pl.BlockSpec((1, 128), full),
        ],
        out_specs=pl.BlockSpec((2, BE, 128), lambda i: (0, i, 0)),
        out_shape=jax.ShapeDtypeStruct((2, EP, 128), jnp.float32),
    )(ag, bg, wr, We2, be2, Wc1, bc1, wc2)


def _xnew(a_ref, xn_ref):
    xn = xn_ref[...]
    deg = xn[:, 3:4]
    upd = jnp.where(deg > 0, xn / jnp.maximum(deg, 1.0), 0.0)
    lane = jax.lax.broadcasted_iota(jnp.int32, xn.shape, 1)
    upd = jnp.where(lane < 3, upd, 0.0)
    return _unpack_ext(a_ref[...]) + upd


def _node_mid_body(h_ref, hn_ref, xn_ref, a_ref,
                   Wn1a_ref, Wn1b_ref, bn1_ref, Wn2_ref, bn2_ref,
                   Wea_ref, Web_ref, be1n_ref,
                   hout_ref, aout_ref, bout_ref):
    x128 = _xnew(a_ref, xn_ref)
    h = h_ref[...]
    z = _silu(jnp.dot(h, Wn1a_ref[...], preferred_element_type=jnp.float32)
              + jnp.dot(hn_ref[...], Wn1b_ref[...],
                        preferred_element_type=jnp.float32)
              + bn1_ref[...])
    hnew = jnp.dot(z, Wn2_ref[...], preferred_element_type=jnp.float32) + bn2_ref[...]
    hout_ref[...] = hnew
    aout_ref[...] = _pack2(jnp.dot(hnew, Wea_ref[...],
                                   preferred_element_type=jnp.float32), x128)
    bout_ref[...] = _pack2(jnp.dot(hnew, Web_ref[...],
                                   preferred_element_type=jnp.float32)
                           + be1n_ref[...], x128)


def _node_mid(h, hn, xn, a, Wn1a, Wn1b, bn1, Wn2, bn2, Wea, Web, be1n):
    grid = (NP // BN,)
    row = lambda i: (i, 0)
    full = lambda i: (0, 0)
    return pl.pallas_call(
        _node_mid_body,
        grid=grid,
        in_specs=[
            pl.BlockSpec((BN, 128), row),
            pl.BlockSpec((BN, 128), row),
            pl.BlockSpec((BN, 128), row),
            pl.BlockSpec((BN, 128), row),
            pl.BlockSpec((128, 128), full),
            pl.BlockSpec((128, 128), full),
            pl.BlockSpec((1, 128), full),
            pl.BlockSpec((128, 128), full),
            pl.BlockSpec((1, 128), full),
            pl.BlockSpec((128, 128), full),
            pl.BlockSpec((128, 128), full),
            pl.BlockSpec((1, 128), full),
        ],
        out_specs=[
            pl.BlockSpec((BN, 128), row),
            pl.BlockSpec((BN, 128), row),
            pl.BlockSpec((BN, 128), row),
        ],
        out_shape=[
            jax.ShapeDtypeStruct((NP, 128), jnp.float32),
            jax.ShapeDtypeStruct((NP, 128), jnp.float32),
            jax.ShapeDtypeStruct((NP, 128), jnp.float32),
        ],
    )(h, hn, xn, a, Wn1a, Wn1b, bn1, Wn2, bn2, Wea, Web, be1n)


def _node_last_body(xn_ref, a_ref, xout_ref):
    xout_ref[...] = _xnew(a_ref, xn_ref)


def _node_last(xn, a):
    grid = (NP // BN,)
    row = lambda i: (i, 0)
    return pl.pallas_call(
        _node_last_body,
        grid=grid,
        in_specs=[
            pl.BlockSpec((BN, 128), row),
            pl.BlockSpec((BN, 128), row),
        ],
        out_specs=pl.BlockSpec((BN, 128), row),
        out_shape=jax.ShapeDtypeStruct((NP, 128), jnp.float32),
    )(xn, a)


def _init_body(h_ref, x128_ref, Wea_ref, Web_ref, be1n_ref,
               aout_ref, bout_ref):
    h = h_ref[...]
    x128 = x128_ref[...]
    aout_ref[...] = _pack2(jnp.dot(h, Wea_ref[...],
                                   preferred_element_type=jnp.float32), x128)
    bout_ref[...] = _pack2(jnp.dot(h, Web_ref[...],
                                   preferred_element_type=jnp.float32)
                           + be1n_ref[...], x128)


def _init_tables(h, x128, Wea, Web, be1n):
    grid = (NP // BN,)
    row = lambda i: (i, 0)
    full = lambda i: (0, 0)
    return pl.pallas_call(
        _init_body,
        grid=grid,
        in_specs=[
            pl.BlockSpec((BN, 128), row),
            pl.BlockSpec((BN, 128), row),
            pl.BlockSpec((128, 128), full),
            pl.BlockSpec((128, 128), full),
            pl.BlockSpec((1, 128), full),
        ],
        out_specs=[
            pl.BlockSpec((BN, 128), row),
            pl.BlockSpec((BN, 128), row),
        ],
        out_shape=[
            jax.ShapeDtypeStruct((NP, 128), jnp.float32),
            jax.ShapeDtypeStruct((NP, 128), jnp.float32),
        ],
    )(h, x128, Wea, Web, be1n)


def kernel(h, x, edge_index, We1, be1, We2, be2, Wc1, bc1, Wc2, Wn1, bn1, Wn2, bn2):
    src = edge_index[0]
    dst = edge_index[1]
    srcp = jnp.concatenate([src, jnp.zeros((EP - E,), src.dtype)])
    dstp = jnp.concatenate([dst, jnp.full((EP - E,), NP - 1, dst.dtype)])
    src3 = srcp.reshape(NWORK, GNCH, GCH)
    dst3 = dstp.reshape(NWORK, GNCH, GCH)
    dst2 = dstp.reshape(NTEC, SNCH, SCH)
    zerorow = jnp.zeros((NP, 128), jnp.float32)
    hp = jnp.pad(h, ((0, NP - N), (0, 0)))
    x128 = jnp.pad(x, ((0, NP - N), (0, 125)))

    a, b = _init_tables(hp, x128, We1[0][:D], We1[0][D:2 * D],
                        be1[0][None, :])
    for l in range(DEPTH):
        ag, bg = _gather_call(a, b, src3, dst3)
        msgall = _edge_stage(ag, bg, We1[l][2 * D][None, :],
                             We2[l], be2[l][None, :],
                             Wc1[l], bc1[l][None, :], Wc2[l][:, 0][None, :])
        hn, xn = _scatter_call(msgall.reshape(2 * EP, 128), dst2, zerorow)
        if l < DEPTH - 1:
            hp, a, b = _node_mid(hp, hn, xn, a,
                                 Wn1[l][:D], Wn1[l][D:], bn1[l][None, :],
                                 Wn2[l], bn2[l][None, :],
                                 We1[l + 1][:D], We1[l + 1][D:2 * D],
                                 be1[l + 1][None, :])
        else:
            xfin = _node_last(xn, a)
    return xfin[:N, :3]


# speedup vs baseline: 2.4490x; 1.0139x over previous
import jax
import jax.numpy as jnp
from jax.experimental import pallas as pl
from jax.experimental.pallas import tpu as pltpu
from jax.experimental.pallas import tpu_sc as plsc

N = 10000
E = 320000
D = 128
DEPTH = 4
NP = 10240
EP = 327680
TW = 256
BE = 1024
BN = 1024

NSC = 2
NTEC = 16
NWORK = NSC * NTEC
EW = EP // NWORK
GCH = 64
GNCH = EW // GCH
GDEPTH = 4
EWS = EP // NTEC
SCH = 128
SNCH = EWS // SCH
SHALF = SNCH // 2
RPT = NP // NTEC

_sc_mesh = plsc.VectorSubcoreMesh(
    core_axis_name="c", subcore_axis_name="s", num_cores=NSC, num_subcores=NTEC)


def _silu(a):
    return a * jax.nn.sigmoid(a)


def _pack2(feat, ext):
    fb = jax.lax.shift_right_logical(
        jax.lax.bitcast_convert_type(feat, jnp.uint32) + jnp.uint32(0x8000),
        jnp.uint32(16))
    eb = jax.lax.bitwise_and(
        jax.lax.bitcast_convert_type(ext, jnp.uint32) + jnp.uint32(0x8000),
        jnp.uint32(0xFFFF0000))
    return jax.lax.bitcast_convert_type(jax.lax.bitwise_or(fb, eb),
                                        jnp.float32)


def _unpack_feat(w):
    b = jax.lax.shift_left(jax.lax.bitcast_convert_type(w, jnp.uint32),
                           jnp.uint32(16))
    return jax.lax.bitcast_convert_type(b, jnp.float32)


def _unpack_ext(w):
    b = jax.lax.bitwise_and(jax.lax.bitcast_convert_type(w, jnp.uint32),
                            jnp.uint32(0xFFFF0000))
    return jax.lax.bitcast_convert_type(b, jnp.float32)


def _gather_body(a_hbm, b_hbm, src_hbm, dst_hbm, ag_out, bg_out,
                 srcv, dstv, bufa0, bufb0, bufa1, bufb1,
                 bufa2, bufb2, bufa3, bufb3,
                 sem0, sem1, sem2, sem3, semw):
    c = jax.lax.axis_index("c")
    s = jax.lax.axis_index("s")
    w = c * NTEC + s
    pltpu.sync_copy(src_hbm.at[w], srcv)
    pltpu.sync_copy(dst_hbm.at[w], dstv)
    bufs = ((bufa0, bufb0, sem0), (bufa1, bufb1, sem1),
            (bufa2, bufb2, sem2), (bufa3, bufb3, sem3))

    def group(t, carry):
        cps = []
        for p in range(GDEPTH):
            j = t * GDEPTH + p
            ba, bb, sm = bufs[p]
            cps.append((pltpu.async_copy(a_hbm.at[srcv.at[j]], ba, sm),
                        pltpu.async_copy(b_hbm.at[dstv.at[j]], bb, sm)))
        outs = []
        for p in range(GDEPTH):
            j = t * GDEPTH + p
            base = pl.multiple_of(w * EW + j * GCH, 8)
            ba, bb, _ = bufs[p]
            cpa, cpb = cps[p]
            cpa.wait()
            cpb.wait()
            outs.append(pltpu.async_copy(ba, ag_out.at[pl.ds(base, GCH)],
                                         semw))
            outs.append(pltpu.async_copy(bb, bg_out.at[pl.ds(base, GCH)],
                                         semw))
        for cp in outs:
            cp.wait()
        return carry

    jax.lax.fori_loop(0, GNCH // GDEPTH, group, 0)


_gather_call = pl.kernel(
    _gather_body,
    out_type=[jax.ShapeDtypeStruct((EP, 128), jnp.float32),
              jax.ShapeDtypeStruct((EP, 128), jnp.float32)],
    mesh=_sc_mesh,
    scratch_types=[
        pltpu.VMEM((GNCH, GCH), jnp.int32),
        pltpu.VMEM((GNCH, GCH), jnp.int32),
        pltpu.VMEM((GCH, 128), jnp.float32),
        pltpu.VMEM((GCH, 128), jnp.float32),
        pltpu.VMEM((GCH, 128), jnp.float32),
        pltpu.VMEM((GCH, 128), jnp.float32),
        pltpu.VMEM((GCH, 128), jnp.float32),
        pltpu.VMEM((GCH, 128), jnp.float32),
        pltpu.VMEM((GCH, 128), jnp.float32),
        pltpu.VMEM((GCH, 128), jnp.float32),
        pltpu.SemaphoreType.DMA,
        pltpu.SemaphoreType.DMA,
        pltpu.SemaphoreType.DMA,
        pltpu.SemaphoreType.DMA,
        pltpu.SemaphoreType.DMA,
    ],
)


def _scatter_body(msgall_hbm, dst_hbm, zeros_hbm, hn_out, xn_out,
                  dstv, mbuf0, mbuf1, acc, sem0, sem1):
    c = jax.lax.axis_index("c")
    s = jax.lax.axis_index("s")
    r0 = s * RPT
    pltpu.sync_copy(zeros_hbm.at[pl.ds(r0, RPT)], acc.at[pl.ds(r0, RPT)])
    plsc.subcore_barrier()
    bufs = ((mbuf0, sem0), (mbuf1, sem1))

    for half in range(2):
        pltpu.sync_copy(dst_hbm.at[s].at[pl.ds(half * SHALF, SHALF)], dstv)

        def pair(t, carry):
            cps = []
            for p in range(2):
                j = t * 2 + p
                base = pl.multiple_of(
                    c * EP + s * EWS + (half * SHALF + j) * SCH, 8)
                mb, sm = bufs[p]
                cps.append(pltpu.async_copy(msgall_hbm.at[pl.ds(base, SCH)],
                                            mb, sm))
            for p in range(2):
                j = t * 2 + p
                mb, _ = bufs[p]
                cps[p].wait()
                pltpu.sync_copy(mb, acc.at[dstv.at[j]], add=True)
            return carry

        jax.lax.fori_loop(0, SHALF // 2, pair, 0)

    plsc.subcore_barrier()

    @pl.when(c == 0)
    def _():
        pltpu.sync_copy(acc.at[pl.ds(r0, RPT)], hn_out.at[pl.ds(r0, RPT)])

    @pl.when(c == 1)
    def _():
        pltpu.sync_copy(acc.at[pl.ds(r0, RPT)], xn_out.at[pl.ds(r0, RPT)])


_scatter_call = pl.kernel(
    _scatter_body,
    out_type=[jax.ShapeDtypeStruct((NP, 128), jnp.float32),
              jax.ShapeDtypeStruct((NP, 128), jnp.float32)],
    mesh=_sc_mesh,
    scratch_types=[
        pltpu.VMEM((SHALF, SCH), jnp.int32),
        pltpu.VMEM((SCH, 128), jnp.float32),
        pltpu.VMEM((SCH, 128), jnp.float32),
        pltpu.VMEM_SHARED((NP, 128), jnp.float32),
        pltpu.SemaphoreType.DMA,
        pltpu.SemaphoreType.DMA,
    ],
)


def _edge_body(ag_ref, bg_ref, wr_ref, We2_ref, be2_ref,
               Wc1_ref, bc1_ref, wc2_ref, msgall_ref):
    ag = ag_ref[...]
    bg = bg_ref[...]
    xdiff = _unpack_ext(ag) - _unpack_ext(bg)
    radial = jnp.sum(xdiff * xdiff, axis=1, keepdims=True)
    inv = 1.0 / (jnp.sqrt(radial) + 1e-30)
    p = _unpack_feat(ag) + _unpack_feat(bg) + radial * wr_ref[...]
    z1 = _silu(p)
    msg_h = _silu(jnp.dot(z1, We2_ref[...],
                          preferred_element_type=jnp.float32) + be2_ref[...])
    t = _silu(jnp.dot(msg_h, Wc1_ref[...],
                      preferred_element_type=jnp.float32) + bc1_ref[...])
    coef = jnp.sum(t * wc2_ref[...], axis=1, keepdims=True)
    lane = jax.lax.broadcasted_iota(jnp.int32, xdiff.shape, 1)
    msgall_ref[0] = msg_h
    msgall_ref[1] = xdiff * (coef * inv) + (lane == 3).astype(jnp.float32)


def _edge_stage(ag, bg, wr, We2, be2, Wc1, bc1, wc2):
    grid = (EP // BE,)
    row = lambda i: (i, 0)
    full = lambda i: (0, 0)
    return pl.pallas_call(
        _edge_body,
        grid=grid,
        in_specs=[
            pl.BlockSpec((BE, 128), row),
            pl.BlockSpec((BE, 128), row),
            pl.BlockSpec((1, 128), full),
            pl.BlockSpec((128, 128), full),
            pl.BlockSpec((1, 128), full),
            pl.BlockSpec((128, 128), full),
            pl.BlockSpec((1, 128), full),
            pl.BlockSpec((1, 128), full),
        ],
        out_specs=pl.BlockSpec((2, BE, 128), lambda i: (0, i, 0)),
        out_shape=jax.ShapeDtypeStruct((2, EP, 128), jnp.float32),
    )(ag, bg, wr, We2, be2, Wc1, bc1, wc2)


def _xnew(a_ref, xn_ref):
    xn = xn_ref[...]
    deg = xn[:, 3:4]
    upd = jnp.where(deg > 0, xn / jnp.maximum(deg, 1.0), 0.0)
    lane = jax.lax.broadcasted_iota(jnp.int32, xn.shape, 1)
    upd = jnp.where(lane < 3, upd, 0.0)
    return _unpack_ext(a_ref[...]) + upd


def _node_mid_body(h_ref, hn_ref, xn_ref, a_ref,
                   Wn1a_ref, Wn1b_ref, bn1_ref, Wn2_ref, bn2_ref,
                   Wea_ref, Web_ref, be1n_ref,
                   hout_ref, aout_ref, bout_ref):
    x128 = _xnew(a_ref, xn_ref)
    h = h_ref[...]
    z = _silu(jnp.dot(h, Wn1a_ref[...], preferred_element_type=jnp.float32)
              + jnp.dot(hn_ref[...], Wn1b_ref[...],
                        preferred_element_type=jnp.float32)
              + bn1_ref[...])
    hnew = jnp.dot(z, Wn2_ref[...], preferred_element_type=jnp.float32) + bn2_ref[...]
    hout_ref[...] = hnew
    aout_ref[...] = _pack2(jnp.dot(hnew, Wea_ref[...],
                                   preferred_element_type=jnp.float32), x128)
    bout_ref[...] = _pack2(jnp.dot(hnew, Web_ref[...],
                                   preferred_element_type=jnp.float32)
                           + be1n_ref[...], x128)


def _node_mid(h, hn, xn, a, Wn1a, Wn1b, bn1, Wn2, bn2, Wea, Web, be1n):
    grid = (NP // BN,)
    row = lambda i: (i, 0)
    full = lambda i: (0, 0)
    return pl.pallas_call(
        _node_mid_body,
        grid=grid,
        in_specs=[
            pl.BlockSpec((BN, 128), row),
            pl.BlockSpec((BN, 128), row),
            pl.BlockSpec((BN, 128), row),
            pl.BlockSpec((BN, 128), row),
            pl.BlockSpec((128, 128), full),
            pl.BlockSpec((128, 128), full),
            pl.BlockSpec((1, 128), full),
            pl.BlockSpec((128, 128), full),
            pl.BlockSpec((1, 128), full),
            pl.BlockSpec((128, 128), full),
            pl.BlockSpec((128, 128), full),
            pl.BlockSpec((1, 128), full),
        ],
        out_specs=[
            pl.BlockSpec((BN, 128), row),
            pl.BlockSpec((BN, 128), row),
            pl.BlockSpec((BN, 128), row),
        ],
        out_shape=[
            jax.ShapeDtypeStruct((NP, 128), jnp.float32),
            jax.ShapeDtypeStruct((NP, 128), jnp.float32),
            jax.ShapeDtypeStruct((NP, 128), jnp.float32),
        ],
    )(h, hn, xn, a, Wn1a, Wn1b, bn1, Wn2, bn2, Wea, Web, be1n)


def _node_last_body(xn_ref, a_ref, xout_ref):
    xout_ref[...] = _xnew(a_ref, xn_ref)


def _node_last(xn, a):
    grid = (NP // BN,)
    row = lambda i: (i, 0)
    return pl.pallas_call(
        _node_last_body,
        grid=grid,
        in_specs=[
            pl.BlockSpec((BN, 128), row),
            pl.BlockSpec((BN, 128), row),
        ],
        out_specs=pl.BlockSpec((BN, 128), row),
        out_shape=jax.ShapeDtypeStruct((NP, 128), jnp.float32),
    )(xn, a)


def _init_body(h_ref, x128_ref, Wea_ref, Web_ref, be1n_ref,
               aout_ref, bout_ref):
    h = h_ref[...]
    x128 = x128_ref[...]
    aout_ref[...] = _pack2(jnp.dot(h, Wea_ref[...],
                                   preferred_element_type=jnp.float32), x128)
    bout_ref[...] = _pack2(jnp.dot(h, Web_ref[...],
                                   preferred_element_type=jnp.float32)
                           + be1n_ref[...], x128)


def _init_tables(h, x128, Wea, Web, be1n):
    grid = (NP // BN,)
    row = lambda i: (i, 0)
    full = lambda i: (0, 0)
    return pl.pallas_call(
        _init_body,
        grid=grid,
        in_specs=[
            pl.BlockSpec((BN, 128), row),
            pl.BlockSpec((BN, 128), row),
            pl.BlockSpec((128, 128), full),
            pl.BlockSpec((128, 128), full),
            pl.BlockSpec((1, 128), full),
        ],
        out_specs=[
            pl.BlockSpec((BN, 128), row),
            pl.BlockSpec((BN, 128), row),
        ],
        out_shape=[
            jax.ShapeDtypeStruct((NP, 128), jnp.float32),
            jax.ShapeDtypeStruct((NP, 128), jnp.float32),
        ],
    )(h, x128, Wea, Web, be1n)


def kernel(h, x, edge_index, We1, be1, We2, be2, Wc1, bc1, Wc2, Wn1, bn1, Wn2, bn2):
    src = edge_index[0]
    dst = edge_index[1]
    srcp = jnp.concatenate([src, jnp.zeros((EP - E,), src.dtype)])
    dstp = jnp.concatenate([dst, jnp.full((EP - E,), NP - 1, dst.dtype)])
    src3 = srcp.reshape(NWORK, GNCH, GCH)
    dst3 = dstp.reshape(NWORK, GNCH, GCH)
    dst2 = dstp.reshape(NTEC, SNCH, SCH)
    zerorow = jnp.zeros((NP, 128), jnp.float32)
    hp = jnp.pad(h, ((0, NP - N), (0, 0)))
    x128 = jnp.pad(x, ((0, NP - N), (0, 125)))

    a, b = _init_tables(hp, x128, We1[0][:D], We1[0][D:2 * D],
                        be1[0][None, :])
    for l in range(DEPTH):
        ag, bg = _gather_call(a, b, src3, dst3)
        msgall = _edge_stage(ag, bg, We1[l][2 * D][None, :],
                             We2[l], be2[l][None, :],
                             Wc1[l], bc1[l][None, :], Wc2[l][:, 0][None, :])
        hn, xn = _scatter_call(msgall.reshape(2 * EP, 128), dst2, zerorow)
        if l < DEPTH - 1:
            hp, a, b = _node_mid(hp, hn, xn, a,
                                 Wn1[l][:D], Wn1[l][D:], bn1[l][None, :],
                                 Wn2[l], bn2[l][None, :],
                                 We1[l + 1][:D], We1[l + 1][D:2 * D],
                                 be1[l + 1][None, :])
        else:
            xfin = _node_last(xn, a)
    return xfin[:N, :3]


# speedup vs baseline: 2.4508x; 1.0007x over previous
import jax
import jax.numpy as jnp
from jax.experimental import pallas as pl
from jax.experimental.pallas import tpu as pltpu
from jax.experimental.pallas import tpu_sc as plsc

N = 10000
E = 320000
D = 128
DEPTH = 4
NP = 10240
EP = 327680
TW = 256
BE = 1024
BN = 1024

NSC = 2
NTEC = 16
NWORK = NSC * NTEC
EW = EP // NWORK
GCH = 64
GNCH = EW // GCH
GDEPTH = 4
EWS = EP // NTEC
SCH = 128
SNCH = EWS // SCH
SHALF = SNCH // 2
RPT = NP // NTEC

import functools


@functools.lru_cache(maxsize=None)
def _sc_mesh():
    return plsc.VectorSubcoreMesh(
        core_axis_name="c", subcore_axis_name="s",
        num_cores=NSC, num_subcores=NTEC)


def _silu(a):
    return a * jax.nn.sigmoid(a)


def _pack2(feat, ext):
    fb = jax.lax.shift_right_logical(
        jax.lax.bitcast_convert_type(feat, jnp.uint32) + jnp.uint32(0x8000),
        jnp.uint32(16))
    eb = jax.lax.bitwise_and(
        jax.lax.bitcast_convert_type(ext, jnp.uint32) + jnp.uint32(0x8000),
        jnp.uint32(0xFFFF0000))
    return jax.lax.bitcast_convert_type(jax.lax.bitwise_or(fb, eb),
                                        jnp.float32)


def _unpack_feat(w):
    b = jax.lax.shift_left(jax.lax.bitcast_convert_type(w, jnp.uint32),
                           jnp.uint32(16))
    return jax.lax.bitcast_convert_type(b, jnp.float32)


def _unpack_ext(w):
    b = jax.lax.bitwise_and(jax.lax.bitcast_convert_type(w, jnp.uint32),
                            jnp.uint32(0xFFFF0000))
    return jax.lax.bitcast_convert_type(b, jnp.float32)


def _gather_body(a_hbm, b_hbm, src_hbm, dst_hbm, ag_out, bg_out,
                 srcv, dstv, bufa0, bufb0, bufa1, bufb1,
                 bufa2, bufb2, bufa3, bufb3,
                 sem0, sem1, sem2, sem3, semw):
    c = jax.lax.axis_index("c")
    s = jax.lax.axis_index("s")
    w = c * NTEC + s
    pltpu.sync_copy(src_hbm.at[w], srcv)
    pltpu.sync_copy(dst_hbm.at[w], dstv)
    bufs = ((bufa0, bufb0, sem0), (bufa1, bufb1, sem1),
            (bufa2, bufb2, sem2), (bufa3, bufb3, sem3))

    def group(t, carry):
        cps = []
        for p in range(GDEPTH):
            j = t * GDEPTH + p
            ba, bb, sm = bufs[p]
            cps.append((pltpu.async_copy(a_hbm.at[srcv.at[j]], ba, sm),
                        pltpu.async_copy(b_hbm.at[dstv.at[j]], bb, sm)))
        outs = []
        for p in range(GDEPTH):
            j = t * GDEPTH + p
            base = pl.multiple_of(w * EW + j * GCH, 8)
            ba, bb, _ = bufs[p]
            cpa, cpb = cps[p]
            cpa.wait()
            cpb.wait()
            outs.append(pltpu.async_copy(ba, ag_out.at[pl.ds(base, GCH)],
                                         semw))
            outs.append(pltpu.async_copy(bb, bg_out.at[pl.ds(base, GCH)],
                                         semw))
        for cp in outs:
            cp.wait()
        return carry

    jax.lax.fori_loop(0, GNCH // GDEPTH, group, 0)


@functools.lru_cache(maxsize=None)
def _gather_call():
    return pl.kernel(
    _gather_body,
    out_type=[jax.ShapeDtypeStruct((EP, 128), jnp.float32),
              jax.ShapeDtypeStruct((EP, 128), jnp.float32)],
    mesh=_sc_mesh(),
    scratch_types=[
        pltpu.VMEM((GNCH, GCH), jnp.int32),
        pltpu.VMEM((GNCH, GCH), jnp.int32),
        pltpu.VMEM((GCH, 128), jnp.float32),
        pltpu.VMEM((GCH, 128), jnp.float32),
        pltpu.VMEM((GCH, 128), jnp.float32),
        pltpu.VMEM((GCH, 128), jnp.float32),
        pltpu.VMEM((GCH, 128), jnp.float32),
        pltpu.VMEM((GCH, 128), jnp.float32),
        pltpu.VMEM((GCH, 128), jnp.float32),
        pltpu.VMEM((GCH, 128), jnp.float32),
        pltpu.SemaphoreType.DMA,
        pltpu.SemaphoreType.DMA,
        pltpu.SemaphoreType.DMA,
        pltpu.SemaphoreType.DMA,
        pltpu.SemaphoreType.DMA,
    ],
)


def _scatter_body(msgall_hbm, dst_hbm, zeros_hbm, hn_out, xn_out,
                  dstv, mbuf0, mbuf1, acc, sem0, sem1):
    c = jax.lax.axis_index("c")
    s = jax.lax.axis_index("s")
    r0 = s * RPT
    pltpu.sync_copy(zeros_hbm.at[pl.ds(r0, RPT)], acc.at[pl.ds(r0, RPT)])
    plsc.subcore_barrier()
    bufs = ((mbuf0, sem0), (mbuf1, sem1))

    for half in range(2):
        pltpu.sync_copy(dst_hbm.at[s].at[pl.ds(half * SHALF, SHALF)], dstv)

        def pair(t, carry):
            cps = []
            for p in range(2):
                j = t * 2 + p
                base = pl.multiple_of(
                    c * EP + s * EWS + (half * SHALF + j) * SCH, 8)
                mb, sm = bufs[p]
                cps.append(pltpu.async_copy(msgall_hbm.at[pl.ds(base, SCH)],
                                            mb, sm))
            for p in range(2):
                j = t * 2 + p
                mb, _ = bufs[p]
                cps[p].wait()
                pltpu.sync_copy(mb, acc.at[dstv.at[j]], add=True)
            return carry

        jax.lax.fori_loop(0, SHALF // 2, pair, 0)

    plsc.subcore_barrier()

    @pl.when(c == 0)
    def _():
        pltpu.sync_copy(acc.at[pl.ds(r0, RPT)], hn_out.at[pl.ds(r0, RPT)])

    @pl.when(c == 1)
    def _():
        pltpu.sync_copy(acc.at[pl.ds(r0, RPT)], xn_out.at[pl.ds(r0, RPT)])


@functools.lru_cache(maxsize=None)
def _scatter_call():
    return pl.kernel(
    _scatter_body,
    out_type=[jax.ShapeDtypeStruct((NP, 128), jnp.float32),
              jax.ShapeDtypeStruct((NP, 128), jnp.float32)],
    mesh=_sc_mesh(),
    scratch_types=[
        pltpu.VMEM((SHALF, SCH), jnp.int32),
        pltpu.VMEM((SCH, 128), jnp.float32),
        pltpu.VMEM((SCH, 128), jnp.float32),
        pltpu.VMEM_SHARED((NP, 128), jnp.float32),
        pltpu.SemaphoreType.DMA,
        pltpu.SemaphoreType.DMA,
    ],
)


def _edge_body(ag_ref, bg_ref, wr_ref, We2_ref, be2_ref,
               Wc1_ref, bc1_ref, wc2_ref, msgall_ref):
    ag = ag_ref[...]
    bg = bg_ref[...]
    xdiff = _unpack_ext(ag) - _unpack_ext(bg)
    radial = jnp.sum(xdiff * xdiff, axis=1, keepdims=True)
    inv = 1.0 / (jnp.sqrt(radial) + 1e-30)
    p = _unpack_feat(ag) + _unpack_feat(bg) + radial * wr_ref[...]
    z1 = _silu(p)
    msg_h = _silu(jnp.dot(z1, We2_ref[...],
                          preferred_element_type=jnp.float32) + be2_ref[...])
    t = _silu(jnp.dot(msg_h, Wc1_ref[...],
                      preferred_element_type=jnp.float32) + bc1_ref[...])
    coef = jnp.sum(t * wc2_ref[...], axis=1, keepdims=True)
    lane = jax.lax.broadcasted_iota(jnp.int32, xdiff.shape, 1)
    msgall_ref[0] = msg_h
    msgall_ref[1] = xdiff * (coef * inv) + (lane == 3).astype(jnp.float32)


def _edge_stage(ag, bg, wr, We2, be2, Wc1, bc1, wc2):
    grid = (EP // BE,)
    row = lambda i: (i, 0)
    full = lambda i: (0, 0)
    return pl.pallas_call(
        _edge_body,
        grid=grid,
        in_specs=[
            pl.BlockSpec((BE, 128), row),
            pl.BlockSpec((BE, 128), row),
            pl.BlockSpec((1, 128), full),
            pl.BlockSpec((128, 128), full),
            pl.BlockSpec((1, 128), full),
            pl.BlockSpec((128, 128), full),
            pl.BlockSpec((1, 128), full),
            pl.BlockSpec((1, 128), full),
        ],
        out_specs=pl.BlockSpec((2, BE, 128), lambda i: (0, i, 0)),
        out_shape=jax.ShapeDtypeStruct((2, EP, 128), jnp.float32),
    )(ag, bg, wr, We2, be2, Wc1, bc1, wc2)


def _xnew(a_ref, xn_ref):
    xn = xn_ref[...]
    deg = xn[:, 3:4]
    upd = jnp.where(deg > 0, xn / jnp.maximum(deg, 1.0), 0.0)
    lane = jax.lax.broadcasted_iota(jnp.int32, xn.shape, 1)
    upd = jnp.where(lane < 3, upd, 0.0)
    return _unpack_ext(a_ref[...]) + upd


def _node_mid_body(h_ref, hn_ref, xn_ref, a_ref,
                   Wn1a_ref, Wn1b_ref, bn1_ref, Wn2_ref, bn2_ref,
                   Wea_ref, Web_ref, be1n_ref,
                   hout_ref, aout_ref, bout_ref):
    x128 = _xnew(a_ref, xn_ref)
    h = h_ref[...]
    z = _silu(jnp.dot(h, Wn1a_ref[...], preferred_element_type=jnp.float32)
              + jnp.dot(hn_ref[...], Wn1b_ref[...],
                        preferred_element_type=jnp.float32)
              + bn1_ref[...])
    hnew = jnp.dot(z, Wn2_ref[...], preferred_element_type=jnp.float32) + bn2_ref[...]
    hout_ref[...] = hnew
    aout_ref[...] = _pack2(jnp.dot(hnew, Wea_ref[...],
                                   preferred_element_type=jnp.float32), x128)
    bout_ref[...] = _pack2(jnp.dot(hnew, Web_ref[...],
                                   preferred_element_type=jnp.float32)
                           + be1n_ref[...], x128)


def _node_mid(h, hn, xn, a, Wn1a, Wn1b, bn1, Wn2, bn2, Wea, Web, be1n):
    grid = (NP // BN,)
    row = lambda i: (i, 0)
    full = lambda i: (0, 0)
    return pl.pallas_call(
        _node_mid_body,
        grid=grid,
        in_specs=[
            pl.BlockSpec((BN, 128), row),
            pl.BlockSpec((BN, 128), row),
            pl.BlockSpec((BN, 128), row),
            pl.BlockSpec((BN, 128), row),
            pl.BlockSpec((128, 128), full),
            pl.BlockSpec((128, 128), full),
            pl.BlockSpec((1, 128), full),
            pl.BlockSpec((128, 128), full),
            pl.BlockSpec((1, 128), full),
            pl.BlockSpec((128, 128), full),
            pl.BlockSpec((128, 128), full),
            pl.BlockSpec((1, 128), full),
        ],
        out_specs=[
            pl.BlockSpec((BN, 128), row),
            pl.BlockSpec((BN, 128), row),
            pl.BlockSpec((BN, 128), row),
        ],
        out_shape=[
            jax.ShapeDtypeStruct((NP, 128), jnp.float32),
            jax.ShapeDtypeStruct((NP, 128), jnp.float32),
            jax.ShapeDtypeStruct((NP, 128), jnp.float32),
        ],
    )(h, hn, xn, a, Wn1a, Wn1b, bn1, Wn2, bn2, Wea, Web, be1n)


def _node_last_body(xn_ref, a_ref, xout_ref):
    xout_ref[...] = _xnew(a_ref, xn_ref)


def _node_last(xn, a):
    grid = (NP // BN,)
    row = lambda i: (i, 0)
    return pl.pallas_call(
        _node_last_body,
        grid=grid,
        in_specs=[
            pl.BlockSpec((BN, 128), row),
            pl.BlockSpec((BN, 128), row),
        ],
        out_specs=pl.BlockSpec((BN, 128), row),
        out_shape=jax.ShapeDtypeStruct((NP, 128), jnp.float32),
    )(xn, a)


def _init_body(h_ref, x128_ref, Wea_ref, Web_ref, be1n_ref,
               aout_ref, bout_ref):
    h = h_ref[...]
    x128 = x128_ref[...]
    aout_ref[...] = _pack2(jnp.dot(h, Wea_ref[...],
                                   preferred_element_type=jnp.float32), x128)
    bout_ref[...] = _pack2(jnp.dot(h, Web_ref[...],
                                   preferred_element_type=jnp.float32)
                           + be1n_ref[...], x128)


def _init_tables(h, x128, Wea, Web, be1n):
    grid = (NP // BN,)
    row = lambda i: (i, 0)
    full = lambda i: (0, 0)
    return pl.pallas_call(
        _init_body,
        grid=grid,
        in_specs=[
            pl.BlockSpec((BN, 128), row),
            pl.BlockSpec((BN, 128), row),
            pl.BlockSpec((128, 128), full),
            pl.BlockSpec((128, 128), full),
            pl.BlockSpec((1, 128), full),
        ],
        out_specs=[
            pl.BlockSpec((BN, 128), row),
            pl.BlockSpec((BN, 128), row),
        ],
        out_shape=[
            jax.ShapeDtypeStruct((NP, 128), jnp.float32),
            jax.ShapeDtypeStruct((NP, 128), jnp.float32),
        ],
    )(h, x128, Wea, Web, be1n)


def kernel(h, x, edge_index, We1, be1, We2, be2, Wc1, bc1, Wc2, Wn1, bn1, Wn2, bn2):
    src = edge_index[0]
    dst = edge_index[1]
    srcp = jnp.concatenate([src, jnp.zeros((EP - E,), src.dtype)])
    dstp = jnp.concatenate([dst, jnp.full((EP - E,), NP - 1, dst.dtype)])
    src3 = srcp.reshape(NWORK, GNCH, GCH)
    dst3 = dstp.reshape(NWORK, GNCH, GCH)
    dst2 = dstp.reshape(NTEC, SNCH, SCH)
    zerorow = jnp.zeros((NP, 128), jnp.float32)
    hp = jnp.pad(h, ((0, NP - N), (0, 0)))
    x128 = jnp.pad(x, ((0, NP - N), (0, 125)))

    a, b = _init_tables(hp, x128, We1[0][:D], We1[0][D:2 * D],
                        be1[0][None, :])
    for l in range(DEPTH):
        ag, bg = _gather_call()(a, b, src3, dst3)
        msgall = _edge_stage(ag, bg, We1[l][2 * D][None, :],
                             We2[l], be2[l][None, :],
                             Wc1[l], bc1[l][None, :], Wc2[l][:, 0][None, :])
        hn, xn = _scatter_call()(msgall.reshape(2 * EP, 128), dst2, zerorow)
        if l < DEPTH - 1:
            hp, a, b = _node_mid(hp, hn, xn, a,
                                 Wn1[l][:D], Wn1[l][D:], bn1[l][None, :],
                                 Wn2[l], bn2[l][None, :],
                                 We1[l + 1][:D], We1[l + 1][D:2 * D],
                                 be1[l + 1][None, :])
        else:
            xfin = _node_last(xn, a)
    return xfin[:N, :3]
